# baseline (speedup 1.0000x reference)
import jax
import jax.numpy as jnp
from jax.experimental import pallas as pl
from jax.experimental.pallas import tpu as pltpu

_K = 1024
_BN = 512
_RC = 64
_LC = 128


def _vq_body(z_ref, e2_hbm, ebf_hbm, esq_hbm, iota_hbm,
             idx_ref, oh_ref, zq_ref,
             e2_s, ebf_s, esq_s, iota_s, sem):
    @pl.when(pl.program_id(0) == 0)
    def _():
        pairs = ((e2_hbm, e2_s), (ebf_hbm, ebf_s),
                 (esq_hbm, esq_s), (iota_hbm, iota_s))
        for src, dst in pairs:
            pltpu.make_async_copy(src, dst, sem).start()
        for src, dst in pairs:
            pltpu.make_async_copy(src, dst, sem).wait()

    e2 = e2_s[...]
    ebf = ebf_s[...]
    esq = esq_s[...]
    iota = iota_s[...]
    z = z_ref[...]
    mm2 = jax.lax.dot_general(
        z, e2, (((1,), (1,)), ((), ())),
        preferred_element_type=jnp.float32,
    )
    zsq = jnp.sum(jnp.square(z), axis=1, keepdims=True)
    for r in range(_BN // _RC):
        r0 = r * _RC
        zsq_r = zsq[r0:r0 + _RC, :]
        bestd = None
        for j in range(_K // _LC):
            j0 = j * _LC
            dj = (zsq_r - mm2[r0:r0 + _RC, j0:j0 + _LC]) + esq[:, j0:j0 + _LC]
            ij = jnp.broadcast_to(iota[:, j0:j0 + _LC], dj.shape)
            if bestd is None:
                bestd, besti = dj, ij
            else:
                lt = dj < bestd
                besti = jnp.where(lt, ij, besti)
                bestd = jnp.where(lt, dj, bestd)
        m = jnp.min(bestd, axis=1, keepdims=True)
        idxf = jnp.min(
            jnp.where(bestd == m, besti, jnp.float32(_K)), axis=1, keepdims=True
        )
        idx_ref[pl.ds(r0, _RC), :] = idxf.astype(jnp.int32)
        oh = jnp.where(iota == idxf, jnp.float32(1.0), jnp.float32(0.0))
        oh_ref[pl.ds(r0, _RC), :] = oh
        zq_ref[pl.ds(r0, _RC), :] = jax.lax.dot_general(
            oh.astype(jnp.bfloat16), ebf, (((1,), (0,)), ((), ())),
            preferred_element_type=jnp.float32,
        )


@jax.jit
def kernel(z_e, embed):
    n, d_ = z_e.shape
    k = embed.shape[0]
    esq = jnp.sum(jnp.square(embed), axis=1)[None, :]
    iota_f = jnp.arange(k, dtype=jnp.float32)[None, :]
    e2 = embed * jnp.float32(2.0)
    ebf = embed.astype(jnp.bfloat16)
    grid = (n // _BN,)
    idx2d, one_hot, z_q = pl.pallas_call(
        _vq_body,
        grid=grid,
        in_specs=[
            pl.BlockSpec((_BN, d_), lambda i: (i, 0)),
            pl.BlockSpec(memory_space=pl.ANY),
            pl.BlockSpec(memory_space=pl.ANY),
            pl.BlockSpec(memory_space=pl.ANY),
            pl.BlockSpec(memory_space=pl.ANY),
        ],
        out_specs=[
            pl.BlockSpec((_BN, 1), lambda i: (i, 0)),
            pl.BlockSpec((_BN, k), lambda i: (i, 0)),
            pl.BlockSpec((_BN, d_), lambda i: (i, 0)),
        ],
        out_shape=[
            jax.ShapeDtypeStruct((n, 1), jnp.int32),
            jax.ShapeDtypeStruct((n, k), jnp.float32),
            jax.ShapeDtypeStruct((n, d_), jnp.float32),
        ],
        scratch_shapes=[
            pltpu.VMEM((k, d_), jnp.float32),
            pltpu.VMEM((k, d_), jnp.bfloat16),
            pltpu.VMEM((1, k), jnp.float32),
            pltpu.VMEM((1, k), jnp.float32),
            pltpu.SemaphoreType.DMA,
        ],
    )(z_e, e2, ebf, esq, iota_f)
    return z_q, idx2d.reshape(n), one_hot


# speedup vs baseline: 1.1603x; 1.1603x over previous
import jax
import jax.numpy as jnp
from jax.experimental import pallas as pl
from jax.experimental.pallas import tpu as pltpu

_K = 1024
_BN = 2048
_RC = 64
_LC = 128


def _vq_body(z_ref, e2_hbm, ebf_hbm, esq_hbm, iota_hbm,
             idx_ref, oh_ref, zq_ref,
             e2_s, ebf_s, esq_s, iota_s, sem):
    @pl.when(pl.program_id(0) == 0)
    def _():
        pairs = ((e2_hbm, e2_s), (ebf_hbm, ebf_s),
                 (esq_hbm, esq_s), (iota_hbm, iota_s))
        for src, dst in pairs:
            pltpu.make_async_copy(src, dst, sem).start()
        for src, dst in pairs:
            pltpu.make_async_copy(src, dst, sem).wait()

    e2 = e2_s[...]
    ebf = ebf_s[...]
    esq = esq_s[...]
    iota = iota_s[...]
    z = z_ref[...]
    mm2 = jax.lax.dot_general(
        z, e2, (((1,), (1,)), ((), ())),
        preferred_element_type=jnp.float32,
    )
    zsq = jnp.sum(jnp.square(z), axis=1, keepdims=True)
    for r in range(_BN // _RC):
        r0 = r * _RC
        zsq_r = zsq[r0:r0 + _RC, :]
        bestd = None
        for j in range(_K // _LC):
            j0 = j * _LC
            dj = (zsq_r - mm2[r0:r0 + _RC, j0:j0 + _LC]) + esq[:, j0:j0 + _LC]
            ij = jnp.broadcast_to(iota[:, j0:j0 + _LC], dj.shape)
            if bestd is None:
                bestd, besti = dj, ij
            else:
                lt = dj < bestd
                besti = jnp.where(lt, ij, besti)
                bestd = jnp.where(lt, dj, bestd)
        m = jnp.min(bestd, axis=1, keepdims=True)
        idxf = jnp.min(
            jnp.where(bestd == m, besti, jnp.float32(_K)), axis=1, keepdims=True
        )
        idx_ref[pl.ds(r0, _RC), :] = idxf.astype(jnp.int32)
        oh = jnp.where(iota == idxf, jnp.float32(1.0), jnp.float32(0.0))
        oh_ref[pl.ds(r0, _RC), :] = oh
        zq_ref[pl.ds(r0, _RC), :] = jax.lax.dot_general(
            oh.astype(jnp.bfloat16), ebf, (((1,), (0,)), ((), ())),
            preferred_element_type=jnp.float32,
        )


@jax.jit
def kernel(z_e, embed):
    n, d_ = z_e.shape
    k = embed.shape[0]
    esq = jnp.sum(jnp.square(embed), axis=1)[None, :]
    iota_f = jnp.arange(k, dtype=jnp.float32)[None, :]
    e2 = embed * jnp.float32(2.0)
    ebf = embed.astype(jnp.bfloat16)
    grid = (n // _BN,)
    idx2d, one_hot, z_q = pl.pallas_call(
        _vq_body,
        grid=grid,
        in_specs=[
            pl.BlockSpec((_BN, d_), lambda i: (i, 0)),
            pl.BlockSpec(memory_space=pl.ANY),
            pl.BlockSpec(memory_space=pl.ANY),
            pl.BlockSpec(memory_space=pl.ANY),
            pl.BlockSpec(memory_space=pl.ANY),
        ],
        out_specs=[
            pl.BlockSpec((_BN, 1), lambda i: (i, 0)),
            pl.BlockSpec((_BN, k), lambda i: (i, 0)),
            pl.BlockSpec((_BN, d_), lambda i: (i, 0)),
        ],
        out_shape=[
            jax.ShapeDtypeStruct((n, 1), jnp.int32),
            jax.ShapeDtypeStruct((n, k), jnp.float32),
            jax.ShapeDtypeStruct((n, d_), jnp.float32),
        ],
        scratch_shapes=[
            pltpu.VMEM((k, d_), jnp.float32),
            pltpu.VMEM((k, d_), jnp.bfloat16),
            pltpu.VMEM((1, k), jnp.float32),
            pltpu.VMEM((1, k), jnp.float32),
            pltpu.SemaphoreType.DMA,
        ],
    )(z_e, e2, ebf, esq, iota_f)
    return z_q, idx2d.reshape(n), one_hot


# speedup vs baseline: 1.2573x; 1.0836x over previous
import jax
import jax.numpy as jnp
from jax.experimental import pallas as pl
from jax.experimental.pallas import tpu as pltpu

_K = 1024
_BN = 1024
_RC = 64
_LC = 128


def _vq_body(z_ref, e2_hbm, ebf_hbm, esq_hbm, iota_hbm,
             idx_ref, oh_ref, zq_ref,
             e2_s, ebf_s, esq_s, iota_s, sem):
    @pl.when(pl.program_id(0) == 0)
    def _():
        pairs = ((e2_hbm, e2_s), (ebf_hbm, ebf_s),
                 (esq_hbm, esq_s), (iota_hbm, iota_s))
        for src, dst in pairs:
            pltpu.make_async_copy(src, dst, sem).start()
        for src, dst in pairs:
            pltpu.make_async_copy(src, dst, sem).wait()

    e2 = e2_s[...]
    ebf = ebf_s[...]
    esq = esq_s[...]
    iota = iota_s[...]
    z = z_ref[...]
    mm2 = jax.lax.dot_general(
        z, e2, (((1,), (1,)), ((), ())),
        preferred_element_type=jnp.float32,
    )
    zsq = jnp.sum(jnp.square(z), axis=1, keepdims=True)
    idx_parts = []
    for r in range(_BN // _RC):
        r0 = r * _RC
        zsq_r = zsq[r0:r0 + _RC, :]
        bestd = None
        for j in range(_K // _LC):
            j0 = j * _LC
            dj = (zsq_r - mm2[r0:r0 + _RC, j0:j0 + _LC]) + esq[:, j0:j0 + _LC]
            ij = jnp.broadcast_to(iota[:, j0:j0 + _LC], dj.shape)
            if bestd is None:
                bestd, besti = dj, ij
            else:
                lt = dj < bestd
                besti = jnp.where(lt, ij, besti)
                bestd = jnp.where(lt, dj, bestd)
        m = jnp.min(bestd, axis=1, keepdims=True)
        idxf = jnp.min(
            jnp.where(bestd == m, besti, jnp.float32(_K)), axis=1, keepdims=True
        )
        idx_parts.append(idxf)
        oh = jnp.where(iota == idxf, jnp.float32(1.0), jnp.float32(0.0))
        oh_ref[pl.ds(r0, _RC), :] = oh
        zq_ref[pl.ds(r0, _RC), :] = jax.lax.dot_general(
            oh.astype(jnp.bfloat16), ebf, (((1,), (0,)), ((), ())),
            preferred_element_type=jnp.float32,
        )
    idx_all = jnp.concatenate(idx_parts, axis=0).reshape(_BN)
    idx_ref[...] = idx_all.astype(jnp.int32)


@jax.jit
def kernel(z_e, embed):
    n, d_ = z_e.shape
    k = embed.shape[0]
    esq = jnp.sum(jnp.square(embed), axis=1)[None, :]
    iota_f = jnp.arange(k, dtype=jnp.float32)[None, :]
    e2 = embed * jnp.float32(2.0)
    ebf = embed.astype(jnp.bfloat16)
    grid = (n // _BN,)
    idx2d, one_hot, z_q = pl.pallas_call(
        _vq_body,
        grid=grid,
        in_specs=[
            pl.BlockSpec((_BN, d_), lambda i: (i, 0)),
            pl.BlockSpec(memory_space=pl.ANY),
            pl.BlockSpec(memory_space=pl.ANY),
            pl.BlockSpec(memory_space=pl.ANY),
            pl.BlockSpec(memory_space=pl.ANY),
        ],
        out_specs=[
            pl.BlockSpec((_BN,), lambda i: (i,)),
            pl.BlockSpec((_BN, k), lambda i: (i, 0)),
            pl.BlockSpec((_BN, d_), lambda i: (i, 0)),
        ],
        out_shape=[
            jax.ShapeDtypeStruct((n,), jnp.int32),
            jax.ShapeDtypeStruct((n, k), jnp.float32),
            jax.ShapeDtypeStruct((n, d_), jnp.float32),
        ],
        scratch_shapes=[
            pltpu.VMEM((k, d_), jnp.float32),
            pltpu.VMEM((k, d_), jnp.bfloat16),
            pltpu.VMEM((1, k), jnp.float32),
            pltpu.VMEM((1, k), jnp.float32),
            pltpu.SemaphoreType.DMA,
        ],
    )(z_e, e2, ebf, esq, iota_f)
    return z_q, idx2d, one_hot
